# baseline (speedup 1.0000x reference)
import functools

import jax
import jax.numpy as jnp
from jax import lax
from jax.experimental import pallas as pl
from jax.experimental.pallas import tpu as pltpu
from jax.experimental.pallas import tpu_sc as plsc

VOCAB = 32000
NUM_NEW = 1024
HIDDEN = 2048
BATCH = 4
SEQ = 8192
TOTAL = BATCH * SEQ

NUM_CORES = 2
NUM_SUBCORES = 16
NW = NUM_CORES * NUM_SUBCORES
PER_W = TOTAL // NW
C = 16
H = C // 2
NCHUNK = PER_W // C
NBUF = 3
PBUF = 2
NSEXT = NCHUNK // 6
NTAIL = NCHUNK - NSEXT * 6
NSPM = NUM_SUBCORES - 1

_mesh = plsc.VectorSubcoreMesh(core_axis_name="c", subcore_axis_name="s")


@functools.partial(
    pl.kernel,
    mesh=_mesh,
    out_type=jax.ShapeDtypeStruct((TOTAL, HIDDEN), jnp.float32),
    scratch_types=[
        pltpu.VMEM((PER_W,), jnp.int32),
        pltpu.VMEM((PER_W,), jnp.int32),
        pltpu.VMEM((C, HIDDEN), jnp.float32),
        pltpu.VMEM((C, HIDDEN), jnp.float32),
        pltpu.VMEM((C, HIDDEN), jnp.float32),
        pltpu.VMEM_SHARED((NSPM * PBUF * H, HIDDEN), jnp.float32),
        pltpu.SemaphoreType.DMA,
        pltpu.SemaphoreType.DMA,
        pltpu.SemaphoreType.DMA,
        pltpu.SemaphoreType.DMA,
        pltpu.SemaphoreType.DMA,
        pltpu.SemaphoreType.DMA,
        pltpu.SemaphoreType.DMA,
        pltpu.SemaphoreType.DMA,
        pltpu.SemaphoreType.DMA,
        pltpu.SemaphoreType.DMA,
        pltpu.SemaphoreType.DMA,
    ],
)
def _encode(ids_hbm, orig_hbm, new_hbm, out_hbm,
            idx_all, gidx_all, rows0, rows1, rows2, bounce,
            gsem0, gsem1, gsem2, wsem0, wsem1, wsem2,
            csem0, csem1, bsem0, bsem1, psem):
    rows = (rows0, rows1, rows2)
    gsem = (gsem0, gsem1, gsem2)
    wsem = (wsem0, wsem1, wsem2)
    csem = (csem0, csem1)
    bsem = (bsem0, bsem1)
    sub = lax.axis_index("s")
    wid = sub * NUM_CORES + lax.axis_index("c")
    base = wid * PER_W
    has_spm = sub < NSPM
    my_spm = jnp.minimum(sub, NSPM - 1) * (PBUF * H)

    pltpu.sync_copy(ids_hbm.at[pl.ds(base, PER_W)], idx_all)

    def clamp_grp(g, carry):
        v = idx_all[pl.ds(g * 16, 16)]
        gidx_all[pl.ds(g * 16, 16)] = jnp.minimum(v, VOCAB - 1)
        return carry

    lax.fori_loop(0, PER_W // 16, clamp_grp, 0)

    def start_gather(ci, b):
        pltpu.async_copy(
            orig_hbm.at[gidx_all.at[pl.ds(ci * C, C)]], rows[b], gsem[b])

    def wait_gather(ci, b):
        pltpu.make_async_copy(
            orig_hbm.at[gidx_all.at[pl.ds(ci * C, C)]], rows[b],
            gsem[b]).wait()

    def spm(p):
        return bounce.at[pl.ds(my_spm + p * H, H)]

    def start_copy(ci, b, p):
        pltpu.async_copy(rows[b].at[pl.ds(0, H)], spm(p), csem[p])

    def wait_copy(ci, b, p):
        pltpu.make_async_copy(rows[b].at[pl.ds(0, H)], spm(p),
                              csem[p]).wait()

    def start_bdma(ci, p):
        pltpu.async_copy(spm(p), out_hbm.at[pl.ds(base + ci * C, H)],
                         bsem[p])

    def wait_bdma(ci, p):
        pltpu.make_async_copy(spm(p), out_hbm.at[pl.ds(base + ci * C, H)],
                              bsem[p]).wait()

    def start_write(ci, b):
        @pl.when(has_spm)
        def _():
            pltpu.async_copy(
                rows[b].at[pl.ds(H, C - H)],
                out_hbm.at[pl.ds(base + ci * C + H, C - H)], wsem[b])

        @pl.when(jnp.logical_not(has_spm))
        def _():
            pltpu.async_copy(
                rows[b], out_hbm.at[pl.ds(base + ci * C, C)], wsem[b])

    def wait_write(ci, b):
        @pl.when(has_spm)
        def _():
            pltpu.make_async_copy(
                rows[b].at[pl.ds(H, C - H)],
                out_hbm.at[pl.ds(base + ci * C + H, C - H)], wsem[b]).wait()

        @pl.when(jnp.logical_not(has_spm))
        def _():
            pltpu.make_async_copy(
                rows[b], out_hbm.at[pl.ds(base + ci * C, C)],
                wsem[b]).wait()

    def patch(ci, b):
        n = jnp.int32(0)
        for g in range(C // 16):
            v = idx_all[pl.ds(ci * C + g * 16, 16)]
            for lane in range(16):
                tid = v[lane]
                n = n + (tid >= VOCAB).astype(jnp.int32)

                @pl.when(tid >= VOCAB)
                def _():
                    pltpu.async_copy(
                        new_hbm.at[pl.ds(tid - VOCAB, 1)],
                        rows[b].at[pl.ds(g * 16 + lane, 1)],
                        psem)

        def drain(i, carry2):
            pltpu.make_async_copy(
                new_hbm.at[pl.ds(0, 1)], rows[b].at[pl.ds(0, 1)],
                psem).wait()
            return carry2

        lax.fori_loop(0, n, drain, 0)

    start_gather(0, 0)

    def step(ci, u6):
        b = u6 % NBUF
        p = u6 % PBUF
        nb = (b + 1) % NBUF

        @pl.when(ci >= 2)
        def _():
            wait_write(ci - 2, nb)

        @pl.when(ci + 1 < NCHUNK)
        def _():
            start_gather(ci + 1, nb)

        wait_gather(ci, b)
        patch(ci, b)
        start_write(ci, b)

        @pl.when(has_spm)
        def _():
            @pl.when(ci >= 2)
            def _():
                wait_bdma(ci - 2, p)

            start_copy(ci, b, p)

            @pl.when(ci >= 1)
            def _():
                wait_copy(ci - 1, (b + NBUF - 1) % NBUF, 1 - p)
                start_bdma(ci - 1, 1 - p)

    def sext_body(t, carry):
        for u6 in range(6):
            step(t * 6 + u6, u6)
        return carry

    lax.fori_loop(0, NSEXT, sext_body, 0)
    for u6 in range(NTAIL):
        step(NSEXT * 6 + u6, u6)

    last = NCHUNK - 1
    wait_write(last - 1, (last - 1) % NBUF)
    wait_write(last, last % NBUF)

    @pl.when(has_spm)
    def _():
        wait_copy(last, last % NBUF, last % PBUF)
        start_bdma(last, last % PBUF)
        wait_bdma(last - 1, (last - 1) % PBUF)
        wait_bdma(last, last % PBUF)


def kernel(input_ids, orig_table, new_table):
    ids = input_ids.reshape(TOTAL).astype(jnp.int32)
    out = _encode(ids, orig_table, new_table)
    return out.reshape(BATCH, SEQ, HIDDEN)
